# Initial kernel scaffold; baseline (speedup 1.0000x reference)
import jax
import jax.numpy as jnp
from jax.experimental import pallas as pl


def kernel(num_frames, height, width, pos_embed):
    raise NotImplementedError("write your pallas kernel here")


# speedup vs baseline: 1.2383x; 1.2383x over previous
import functools

import jax
import jax.numpy as jnp
from jax import lax
from jax.experimental import pallas as pl
from jax.experimental.pallas import tpu as pltpu
from jax.experimental.pallas import tpu_sc as plsc

_NUM_POS = 2304
_HIDDEN = 1152
_MERGE = 2
_GRID = 48
_F = 4
_H = 128
_W = 128
_ROWS = _H * _W
_NW = 32
_RPW = _ROWS // _NW
_B = 16
_NCH = _RPW // _B
_NG = _HIDDEN // 32


def _linspace(stop, num, num_static):
    div = (jnp.asarray(num) - 1).astype(jnp.float32)
    delta = jnp.float32(stop) / div
    body = lax.iota(jnp.float32, num_static - 1) * delta
    return jnp.concatenate([body, jnp.full((1,), stop, dtype=jnp.float32)])


def _sc_body(table_hbm, i0_hbm, i1_hbm, i2_hbm, i3_hbm,
             wall_hbm, c_hbm, out_hbm,
             i0v, i1v, i2v, i3v, wallv, cv,
             r0, r1, r2, r3, ov, sem, osem):
    wid = lax.axis_index("s") * 2 + lax.axis_index("c")
    base = wid * _RPW
    pltpu.sync_copy(i0_hbm.at[pl.ds(base, _RPW)], i0v)
    pltpu.sync_copy(i1_hbm.at[pl.ds(base, _RPW)], i1v)
    pltpu.sync_copy(i2_hbm.at[pl.ds(base, _RPW)], i2v)
    pltpu.sync_copy(i3_hbm.at[pl.ds(base, _RPW)], i3v)
    pltpu.sync_copy(wall_hbm.at[pl.ds(base * 4, _RPW * 4)],
                    wallv.at[pl.ds(0, _RPW * 4)])
    pltpu.sync_copy(c_hbm, cv)

    def chunk_body(ch, _):
        off = ch * _B
        g0 = pltpu.async_copy(table_hbm.at[i0v[pl.ds(off, _B)]], r0, sem)
        g1 = pltpu.async_copy(table_hbm.at[i1v[pl.ds(off, _B)]], r1, sem)
        g2 = pltpu.async_copy(table_hbm.at[i2v[pl.ds(off, _B)]], r2, sem)
        g3 = pltpu.async_copy(table_hbm.at[i3v[pl.ds(off, _B)]], r3, sem)
        g0.wait()
        g1.wait()
        g2.wait()
        g3.wait()
        cvec = cv[...]

        def pos_body(p, _):
            wq = wallv[pl.ds((off + p) * 4, 16)]
            a0 = wq[0]
            a1 = wq[1]
            a2 = wq[2]
            a3 = wq[3]

            def grp_body(g, _):
                ce = (a0 * r0[p, pl.ds(g * 32, 16)]
                      + a1 * r1[p, pl.ds(g * 32, 16)]
                      + a2 * r2[p, pl.ds(g * 32, 16)]
                      + a3 * r3[p, pl.ds(g * 32, 16)] + cvec)
                co = (a0 * r0[p, pl.ds(g * 32 + 16, 16)]
                      + a1 * r1[p, pl.ds(g * 32 + 16, 16)]
                      + a2 * r2[p, pl.ds(g * 32 + 16, 16)]
                      + a3 * r3[p, pl.ds(g * 32 + 16, 16)] + cvec)
                ov[p, pl.ds(g * 32, 32)] = plsc.pack(
                    ce, co, format=plsc.PackFormat.INTERLEAVED)
                return ()

            lax.fori_loop(0, _NG, grp_body, ())
            return ()

        lax.fori_loop(0, _B, pos_body, ())
        s0 = pltpu.async_copy(ov, out_hbm.at[pl.ds(base + off, _B)], osem)
        s1 = pltpu.async_copy(ov, out_hbm.at[pl.ds(_ROWS + base + off, _B)], osem)
        s2 = pltpu.async_copy(ov, out_hbm.at[pl.ds(2 * _ROWS + base + off, _B)], osem)
        s3 = pltpu.async_copy(ov, out_hbm.at[pl.ds(3 * _ROWS + base + off, _B)], osem)
        s0.wait()
        s1.wait()
        s2.wait()
        s3.wait()
        return ()

    lax.fori_loop(0, _NCH, chunk_body, ())


@functools.partial(
    pl.kernel,
    out_type=jax.ShapeDtypeStruct((_F * _ROWS, _HIDDEN), jnp.bfloat16),
    mesh=plsc.VectorSubcoreMesh(core_axis_name="c", subcore_axis_name="s"),
    compiler_params=pltpu.CompilerParams(needs_layout_passes=False),
    scratch_types=[
        pltpu.VMEM((_RPW,), jnp.int32),
        pltpu.VMEM((_RPW,), jnp.int32),
        pltpu.VMEM((_RPW,), jnp.int32),
        pltpu.VMEM((_RPW,), jnp.int32),
        pltpu.VMEM((_RPW * 4 + 16,), jnp.float32),
        pltpu.VMEM((16,), jnp.float32),
        pltpu.VMEM((_B, _HIDDEN), jnp.float32),
        pltpu.VMEM((_B, _HIDDEN), jnp.float32),
        pltpu.VMEM((_B, _HIDDEN), jnp.float32),
        pltpu.VMEM((_B, _HIDDEN), jnp.float32),
        pltpu.VMEM((_B, _HIDDEN), jnp.bfloat16),
        pltpu.SemaphoreType.DMA,
        pltpu.SemaphoreType.DMA,
    ],
)
def _sc_interp(table_hbm, i0_hbm, i1_hbm, i2_hbm, i3_hbm,
               wall_hbm, c_hbm, out_hbm,
               i0v, i1v, i2v, i3v, wallv, cv,
               r0, r1, r2, r3, ov, sem, osem):
    _sc_body(table_hbm, i0_hbm, i1_hbm, i2_hbm, i3_hbm,
             wall_hbm, c_hbm, out_hbm,
             i0v, i1v, i2v, i3v, wallv, cv,
             r0, r1, r2, r3, ov, sem, osem)


def kernel(num_frames, height, width, pos_embed):
    h_idxs = _linspace(_GRID - 1, height, _H)
    w_idxs = _linspace(_GRID - 1, width, _W)
    hf = jnp.floor(h_idxs).astype(jnp.int32)
    wf = jnp.floor(w_idxs).astype(jnp.int32)
    hc = jnp.minimum(hf + 1, _GRID - 1)
    wc = jnp.minimum(wf + 1, _GRID - 1)
    dh = h_idxs - hf
    dw = w_idxs - wf

    r = jnp.arange(_ROWS)
    m = r // (_MERGE * _MERGE * (_W // _MERGE))
    rem = r % (_MERGE * _MERGE * (_W // _MERGE))
    n = rem // (_MERGE * _MERGE)
    ij = rem % (_MERGE * _MERGE)
    i = ij // _MERGE
    j = ij % _MERGE
    h = _MERGE * m + i
    w = _MERGE * n + j

    hfr = hf[h]
    hcr = hc[h]
    wfr = wf[w]
    wcr = wc[w]
    dhr = dh[h]
    dwr = dw[w]
    i0 = hfr * _GRID + wfr
    i1 = hfr * _GRID + wcr
    i2 = hcr * _GRID + wfr
    i3 = hcr * _GRID + wcr
    w0 = (1 - dhr) * (1 - dwr)
    w1 = (1 - dhr) * dwr
    w2 = dhr * (1 - dwr)
    w3 = dhr * dwr
    wall = jnp.stack([w0, w1, w2, w3], axis=1).reshape(-1)

    g = jnp.arange(_HIDDEN)
    grp = g // 32
    lane = g % 32
    colperm = grp * 32 + jnp.where(lane < 16, 2 * lane, 2 * (lane - 16) + 1)
    table_p = pos_embed[:, colperm]

    cvec = jnp.full((16,), (jnp.asarray(num_frames) - _F), dtype=jnp.float32)

    return _sc_interp(table_p, i0, i1, i2, i3, wall, cvec)


# speedup vs baseline: 1.2411x; 1.0022x over previous
import functools

import jax
import jax.numpy as jnp
from jax import lax
from jax.experimental import pallas as pl
from jax.experimental.pallas import tpu as pltpu
from jax.experimental.pallas import tpu_sc as plsc

_NUM_POS = 2304
_HIDDEN = 1152
_MERGE = 2
_GRID = 48
_F = 4
_H = 128
_W = 128
_ROWS = _H * _W
_NW = 32
_RPW = _ROWS // _NW
_B = 16
_NCH = _RPW // _B
_NG = _HIDDEN // 32


def _linspace(stop, num, num_static):
    div = (jnp.asarray(num) - 1).astype(jnp.float32)
    delta = jnp.float32(stop) / div
    body = lax.iota(jnp.float32, num_static - 1) * delta
    return jnp.concatenate([body, jnp.full((1,), stop, dtype=jnp.float32)])


def _sc_body(table_hbm, i0_hbm, i1_hbm, i2_hbm, i3_hbm,
             wall_hbm, c_hbm, out_hbm,
             i0v, i1v, i2v, i3v, wallv, cv,
             r0, r1, r2, r3, ov, sem, osem):
    wid = lax.axis_index("s") * 2 + lax.axis_index("c")
    base = wid * _RPW
    pltpu.sync_copy(i0_hbm.at[pl.ds(base, _RPW)], i0v)
    pltpu.sync_copy(i1_hbm.at[pl.ds(base, _RPW)], i1v)
    pltpu.sync_copy(i2_hbm.at[pl.ds(base, _RPW)], i2v)
    pltpu.sync_copy(i3_hbm.at[pl.ds(base, _RPW)], i3v)
    pltpu.sync_copy(wall_hbm.at[pl.ds(base * 4, _RPW * 4)],
                    wallv.at[pl.ds(0, _RPW * 4)])
    pltpu.sync_copy(c_hbm, cv)

    def chunk_body(ch, _):
        off = ch * _B
        g0 = pltpu.async_copy(table_hbm.at[i0v[pl.ds(off, _B)]], r0, sem)
        g1 = pltpu.async_copy(table_hbm.at[i1v[pl.ds(off, _B)]], r1, sem)
        g2 = pltpu.async_copy(table_hbm.at[i2v[pl.ds(off, _B)]], r2, sem)
        g3 = pltpu.async_copy(table_hbm.at[i3v[pl.ds(off, _B)]], r3, sem)
        g0.wait()
        g1.wait()
        g2.wait()
        g3.wait()
        cvec = cv[...]

        def pos_body(p, _):
            wq = wallv[pl.ds((off + p) * 4, 16)]
            a0 = wq[0]
            a1 = wq[1]
            a2 = wq[2]
            a3 = wq[3]

            for g in range(_NG):
                ce = (a0 * r0[p, pl.ds(g * 32, 16)]
                      + a1 * r1[p, pl.ds(g * 32, 16)]
                      + a2 * r2[p, pl.ds(g * 32, 16)]
                      + a3 * r3[p, pl.ds(g * 32, 16)] + cvec)
                co = (a0 * r0[p, pl.ds(g * 32 + 16, 16)]
                      + a1 * r1[p, pl.ds(g * 32 + 16, 16)]
                      + a2 * r2[p, pl.ds(g * 32 + 16, 16)]
                      + a3 * r3[p, pl.ds(g * 32 + 16, 16)] + cvec)
                ov[p, pl.ds(g * 32, 32)] = plsc.pack(
                    ce, co, format=plsc.PackFormat.INTERLEAVED)
            return ()

        lax.fori_loop(0, _B, pos_body, ())
        s0 = pltpu.async_copy(ov, out_hbm.at[pl.ds(base + off, _B)], osem)
        s1 = pltpu.async_copy(ov, out_hbm.at[pl.ds(_ROWS + base + off, _B)], osem)
        s2 = pltpu.async_copy(ov, out_hbm.at[pl.ds(2 * _ROWS + base + off, _B)], osem)
        s3 = pltpu.async_copy(ov, out_hbm.at[pl.ds(3 * _ROWS + base + off, _B)], osem)
        s0.wait()
        s1.wait()
        s2.wait()
        s3.wait()
        return ()

    lax.fori_loop(0, _NCH, chunk_body, ())


@functools.partial(
    pl.kernel,
    out_type=jax.ShapeDtypeStruct((_F * _ROWS, _HIDDEN), jnp.bfloat16),
    mesh=plsc.VectorSubcoreMesh(core_axis_name="c", subcore_axis_name="s"),
    compiler_params=pltpu.CompilerParams(needs_layout_passes=False),
    scratch_types=[
        pltpu.VMEM((_RPW,), jnp.int32),
        pltpu.VMEM((_RPW,), jnp.int32),
        pltpu.VMEM((_RPW,), jnp.int32),
        pltpu.VMEM((_RPW,), jnp.int32),
        pltpu.VMEM((_RPW * 4 + 16,), jnp.float32),
        pltpu.VMEM((16,), jnp.float32),
        pltpu.VMEM((_B, _HIDDEN), jnp.float32),
        pltpu.VMEM((_B, _HIDDEN), jnp.float32),
        pltpu.VMEM((_B, _HIDDEN), jnp.float32),
        pltpu.VMEM((_B, _HIDDEN), jnp.float32),
        pltpu.VMEM((_B, _HIDDEN), jnp.bfloat16),
        pltpu.SemaphoreType.DMA,
        pltpu.SemaphoreType.DMA,
    ],
)
def _sc_interp(table_hbm, i0_hbm, i1_hbm, i2_hbm, i3_hbm,
               wall_hbm, c_hbm, out_hbm,
               i0v, i1v, i2v, i3v, wallv, cv,
               r0, r1, r2, r3, ov, sem, osem):
    _sc_body(table_hbm, i0_hbm, i1_hbm, i2_hbm, i3_hbm,
             wall_hbm, c_hbm, out_hbm,
             i0v, i1v, i2v, i3v, wallv, cv,
             r0, r1, r2, r3, ov, sem, osem)


def kernel(num_frames, height, width, pos_embed):
    h_idxs = _linspace(_GRID - 1, height, _H)
    w_idxs = _linspace(_GRID - 1, width, _W)
    hf = jnp.floor(h_idxs).astype(jnp.int32)
    wf = jnp.floor(w_idxs).astype(jnp.int32)
    hc = jnp.minimum(hf + 1, _GRID - 1)
    wc = jnp.minimum(wf + 1, _GRID - 1)
    dh = h_idxs - hf
    dw = w_idxs - wf

    r = jnp.arange(_ROWS)
    m = r // (_MERGE * _MERGE * (_W // _MERGE))
    rem = r % (_MERGE * _MERGE * (_W // _MERGE))
    n = rem // (_MERGE * _MERGE)
    ij = rem % (_MERGE * _MERGE)
    i = ij // _MERGE
    j = ij % _MERGE
    h = _MERGE * m + i
    w = _MERGE * n + j

    hfr = hf[h]
    hcr = hc[h]
    wfr = wf[w]
    wcr = wc[w]
    dhr = dh[h]
    dwr = dw[w]
    i0 = hfr * _GRID + wfr
    i1 = hfr * _GRID + wcr
    i2 = hcr * _GRID + wfr
    i3 = hcr * _GRID + wcr
    w0 = (1 - dhr) * (1 - dwr)
    w1 = (1 - dhr) * dwr
    w2 = dhr * (1 - dwr)
    w3 = dhr * dwr
    wall = jnp.stack([w0, w1, w2, w3], axis=1).reshape(-1)

    g = jnp.arange(_HIDDEN)
    grp = g // 32
    lane = g % 32
    colperm = grp * 32 + jnp.where(lane < 16, 2 * lane, 2 * (lane - 16) + 1)
    table_p = pos_embed[:, colperm]

    cvec = jnp.full((16,), (jnp.asarray(num_frames) - _F), dtype=jnp.float32)

    return _sc_interp(table_p, i0, i1, i2, i3, wall, cvec)
